# baseline (speedup 1.0000x reference)
import functools

import jax
import jax.numpy as jnp
from jax import lax
from jax.experimental import pallas as pl
from jax.experimental.pallas import tpu as pltpu
from jax.experimental.pallas import tpu_sc as plsc

_NC = 2
_NS = 16
_NW = _NC * _NS
_C = 64
_NBUF = 8
_K = 4


def _gather_kernel(n, embed, nchunk):
    mesh = plsc.VectorSubcoreMesh(core_axis_name="c", subcore_axis_name="s")
    assert (nchunk - _NBUF) % _NBUF == 0

    @functools.partial(
        pl.kernel,
        out_type=jax.ShapeDtypeStruct((n, embed), jnp.float32),
        mesh=mesh,
        scratch_types=[
            pltpu.VMEM((nchunk, _C), jnp.int32),
            pltpu.VMEM((_NBUF, _C, embed), jnp.float32),
            pltpu.SemaphoreType.DMA((_NBUF,)),
            pltpu.SemaphoreType.DMA((_NBUF,)),
        ],
    )
    def body(idx_hbm, table_hbm, out_hbm, idx_v, rows_v, sem_g, sem_o):
        wid = lax.axis_index("s") * _NC + lax.axis_index("c")
        row0 = wid * (nchunk * _C)

        pltpu.sync_copy(idx_hbm.at[wid], idx_v)

        def start_gather(j, b):
            pltpu.async_copy(table_hbm.at[idx_v.at[j]], rows_v.at[b],
                             sem_g.at[b])

        def wait_gather(j, b):
            pltpu.make_async_copy(table_hbm.at[idx_v.at[j]], rows_v.at[b],
                                  sem_g.at[b]).wait()

        def start_copyout(j, b):
            pltpu.async_copy(rows_v.at[b],
                             out_hbm.at[pl.ds(row0 + j * _C, _C)],
                             sem_o.at[b])

        def wait_copyout(j, b):
            pltpu.make_async_copy(rows_v.at[b],
                                  out_hbm.at[pl.ds(row0 + j * _C, _C)],
                                  sem_o.at[b]).wait()

        for j in range(_K):
            start_gather(j, j)

        for j in range(_NBUF - _K):
            start_gather(j + _K, j + _K)
            wait_gather(j, j)
            start_copyout(j, j)

        @pl.loop(0, (nchunk - _NBUF) // _NBUF)
        def _(s):
            for i in range(_NBUF):
                j = (_NBUF - _K) + s * _NBUF + i
                bk = i
                b = (_NBUF - _K + i) % _NBUF
                wait_copyout(j + _K - _NBUF, bk)
                start_gather(j + _K, bk)
                wait_gather(j, b)
                start_copyout(j, b)

        for t in range(_K):
            j = nchunk - _K + t
            b = j % _NBUF
            wait_gather(j, b)
            start_copyout(j, b)

        for b in range(_NBUF):
            j = nchunk - _NBUF + b
            wait_copyout(j, j % _NBUF)

    return body


def kernel(inputs, ids, values):
    del ids
    n = inputs.shape[0]
    embed = values.shape[1]
    nchunk = n // (_NW * _C)
    idx = inputs.reshape(_NW, nchunk, _C)
    return _gather_kernel(n, embed, nchunk)(idx, values)


# speedup vs baseline: 1.0019x; 1.0019x over previous
import functools

import jax
import jax.numpy as jnp
from jax import lax
from jax.experimental import pallas as pl
from jax.experimental.pallas import tpu as pltpu
from jax.experimental.pallas import tpu_sc as plsc

_NC = 2
_NS = 16
_NW = _NC * _NS
_C = 64
_NBUF = 8
_K = 6


def _gather_kernel(n, embed, nchunk):
    mesh = plsc.VectorSubcoreMesh(core_axis_name="c", subcore_axis_name="s")
    assert (nchunk - _NBUF) % _NBUF == 0

    @functools.partial(
        pl.kernel,
        out_type=jax.ShapeDtypeStruct((n, embed), jnp.float32),
        mesh=mesh,
        scratch_types=[
            pltpu.VMEM((nchunk, _C), jnp.int32),
            pltpu.VMEM((_NBUF, _C, embed), jnp.float32),
            pltpu.SemaphoreType.DMA((_NBUF,)),
            pltpu.SemaphoreType.DMA((_NBUF,)),
        ],
    )
    def body(idx_hbm, table_hbm, out_hbm, idx_v, rows_v, sem_g, sem_o):
        wid = lax.axis_index("s") * _NC + lax.axis_index("c")
        row0 = wid * (nchunk * _C)

        pltpu.sync_copy(idx_hbm.at[wid], idx_v)

        def start_gather(j, b):
            pltpu.async_copy(table_hbm.at[idx_v.at[j]], rows_v.at[b],
                             sem_g.at[b])

        def wait_gather(j, b):
            pltpu.make_async_copy(table_hbm.at[idx_v.at[j]], rows_v.at[b],
                                  sem_g.at[b]).wait()

        def start_copyout(j, b):
            pltpu.async_copy(rows_v.at[b],
                             out_hbm.at[pl.ds(row0 + j * _C, _C)],
                             sem_o.at[b])

        def wait_copyout(j, b):
            pltpu.make_async_copy(rows_v.at[b],
                                  out_hbm.at[pl.ds(row0 + j * _C, _C)],
                                  sem_o.at[b]).wait()

        for j in range(_K):
            start_gather(j, j)

        for j in range(_NBUF - _K):
            start_gather(j + _K, j + _K)
            wait_gather(j, j)
            start_copyout(j, j)

        @pl.loop(0, (nchunk - _NBUF) // _NBUF)
        def _(s):
            for i in range(_NBUF):
                j = (_NBUF - _K) + s * _NBUF + i
                bk = i
                b = (_NBUF - _K + i) % _NBUF
                wait_copyout(j + _K - _NBUF, bk)
                start_gather(j + _K, bk)
                wait_gather(j, b)
                start_copyout(j, b)

        for t in range(_K):
            j = nchunk - _K + t
            b = j % _NBUF
            wait_gather(j, b)
            start_copyout(j, b)

        for b in range(_NBUF):
            j = nchunk - _NBUF + b
            wait_copyout(j, j % _NBUF)

    return body


def kernel(inputs, ids, values):
    del ids
    n = inputs.shape[0]
    embed = values.shape[1]
    nchunk = n // (_NW * _C)
    idx = inputs.reshape(_NW, nchunk, _C)
    return _gather_kernel(n, embed, nchunk)(idx, values)
